# baseline (speedup 1.0000x reference)
import functools

import jax
import jax.numpy as jnp
from jax.experimental import pallas as pl
from jax.experimental.pallas import tpu as pltpu

B, S, D = 1, 2048, 768
E, K, H = 8, 2, 3072
T = B * S
BT = 256
NT = T // BT


def _gating_kernel(tok_ref, wg_ref, gate_ref):
    logits = jnp.dot(tok_ref[...], wg_ref[...],
                     preferred_element_type=jnp.float32)
    m = jnp.max(logits, axis=-1, keepdims=True)
    ex = jnp.exp(logits - m)
    probs = ex / jnp.sum(ex, axis=-1, keepdims=True)

    eidx = jax.lax.broadcasted_iota(jnp.int32, (T, E), 1)
    big = jnp.int32(E + 1)

    v1 = jnp.max(probs, axis=-1, keepdims=True)
    i1 = jnp.min(jnp.where(probs == v1, eidx, big), axis=-1, keepdims=True)
    probs2 = jnp.where(eidx == i1, -jnp.inf, probs)
    v2 = jnp.max(probs2, axis=-1, keepdims=True)
    i2 = jnp.min(jnp.where(probs2 == v2, eidx, big), axis=-1, keepdims=True)

    s = v1 + v2
    gate_ref[...] = (jnp.where(eidx == i1, v1 / s, 0.0)
                     + jnp.where(eidx == i2, v2 / s, 0.0))


def _ffn_kernel(tok_ref, w1_ref, b1_ref, w2_ref, b2_ref, gate_ref, y_ref):
    e = pl.program_id(0)
    t = pl.program_id(1)

    xb = tok_ref[pl.ds(t * BT, BT), :]
    h = jnp.dot(xb, w1_ref[0], preferred_element_type=jnp.float32)
    h = h + b1_ref[0]
    a = jax.nn.gelu(h).astype(jnp.bfloat16)
    o = jnp.dot(a, w2_ref[0], preferred_element_type=jnp.float32)
    o = o + b2_ref[0]

    gb = gate_ref[pl.ds(t * BT, BT), :]
    eidx = jax.lax.broadcasted_iota(jnp.int32, (BT, E), 1)
    g = jnp.sum(jnp.where(eidx == e, gb, 0.0), axis=1, keepdims=True)
    contrib = o * g

    @pl.when(e == 0)
    def _():
        y_ref[pl.ds(t * BT, BT), :] = contrib

    @pl.when(e > 0)
    def _():
        y_ref[pl.ds(t * BT, BT), :] = y_ref[pl.ds(t * BT, BT), :] + contrib


def kernel(x, Wg, W1, b1, W2, b2):
    tok = x.reshape(T, D)

    gate = pl.pallas_call(
        _gating_kernel,
        out_shape=jax.ShapeDtypeStruct((T, E), jnp.float32),
    )(tok, Wg)

    y = pl.pallas_call(
        _ffn_kernel,
        grid=(E, NT),
        in_specs=[
            pl.BlockSpec((T, D), lambda e, t: (0, 0)),
            pl.BlockSpec((1, D, H), lambda e, t: (e, 0, 0)),
            pl.BlockSpec((1, 1, H), lambda e, t: (e, 0, 0)),
            pl.BlockSpec((1, H, D), lambda e, t: (e, 0, 0)),
            pl.BlockSpec((1, 1, D), lambda e, t: (e, 0, 0)),
            pl.BlockSpec((T, E), lambda e, t: (0, 0)),
        ],
        out_specs=pl.BlockSpec((T, D), lambda e, t: (0, 0)),
        out_shape=jax.ShapeDtypeStruct((T, D), jnp.float32),
    )(tok.astype(jnp.bfloat16), W1.astype(jnp.bfloat16),
      b1.reshape(E, 1, H), W2.astype(jnp.bfloat16),
      b2.reshape(E, 1, D), gate)

    return y.reshape(B, S, D)


# speedup vs baseline: 1.1293x; 1.1293x over previous
import functools

import jax
import jax.numpy as jnp
from jax.experimental import pallas as pl
from jax.experimental.pallas import tpu as pltpu

B, S, D = 1, 2048, 768
E, K, H = 8, 2, 3072
T = B * S

BBT = 128
NSLOT = T * K
NPAD = NSLOT + E * BBT
NBLK = NPAD // BBT


def _gating_kernel(tok_ref, wg_ref, topi_ref, topw_ref):
    logits = jnp.dot(tok_ref[...], wg_ref[...],
                     preferred_element_type=jnp.float32)
    m = jnp.max(logits, axis=-1, keepdims=True)
    ex = jnp.exp(logits - m)
    probs = ex / jnp.sum(ex, axis=-1, keepdims=True)

    eidx = jax.lax.broadcasted_iota(jnp.int32, (T, E), 1)
    big = jnp.int32(E + 1)

    v1 = jnp.max(probs, axis=-1, keepdims=True)
    i1 = jnp.min(jnp.where(probs == v1, eidx, big), axis=-1, keepdims=True)
    probs2 = jnp.where(eidx == i1, -jnp.inf, probs)
    v2 = jnp.max(probs2, axis=-1, keepdims=True)
    i2 = jnp.min(jnp.where(probs2 == v2, eidx, big), axis=-1, keepdims=True)

    s = v1 + v2
    topi_ref[...] = jnp.concatenate([i1, i2], axis=1)
    topw_ref[...] = jnp.concatenate([v1 / s, v2 / s], axis=1)


def _ffn_grouped_kernel(eid_ref, valid_ref, xg_ref, w1_ref, b1_ref,
                        w2_ref, b2_ref, yg_ref):
    b = pl.program_id(0)

    @pl.when(valid_ref[b] > 0)
    def _():
        h = jnp.dot(xg_ref[...], w1_ref[0],
                    preferred_element_type=jnp.float32)
        h = h + b1_ref[0]
        a = jax.nn.gelu(h)
        o = jnp.dot(a, w2_ref[0], preferred_element_type=jnp.float32)
        yg_ref[...] = o + b2_ref[0]

    @pl.when(valid_ref[b] <= 0)
    def _():
        yg_ref[...] = jnp.zeros_like(yg_ref)


def _dispatch_host(topi):
    eids = topi.reshape(NSLOT)
    counts = jnp.bincount(eids, length=E).astype(jnp.int32)
    padded = ((counts + BBT - 1) // BBT) * BBT
    off = jnp.concatenate([jnp.zeros(1, jnp.int32),
                           jnp.cumsum(padded)[:-1].astype(jnp.int32)])
    gstart = jnp.concatenate([jnp.zeros(1, jnp.int32),
                              jnp.cumsum(counts)[:-1].astype(jnp.int32)])

    order = jnp.argsort(eids, stable=True)
    g = eids[order]
    slot_sorted = off[g] + (jnp.arange(NSLOT, dtype=jnp.int32) - gstart[g])
    slot = jnp.zeros(NSLOT, jnp.int32).at[order].set(slot_sorted)

    ridx = jnp.zeros(NPAD, jnp.int32).at[slot_sorted].set(
        (order // K).astype(jnp.int32))

    bounds = jnp.cumsum(padded).astype(jnp.int32)
    bstart = jnp.arange(NBLK, dtype=jnp.int32) * BBT
    blk_eid = jnp.minimum(
        jnp.sum(bounds[None, :] <= bstart[:, None], axis=1), E - 1
    ).astype(jnp.int32)
    gend_valid = off + counts
    blk_valid = jnp.clip(gend_valid[blk_eid] - bstart, 0, BBT)

    return ridx, slot.reshape(T, K), blk_eid, blk_valid


def kernel(x, Wg, W1, b1, W2, b2):
    tok = x.reshape(T, D)

    topi, topw = pl.pallas_call(
        _gating_kernel,
        out_shape=(jax.ShapeDtypeStruct((T, K), jnp.int32),
                   jax.ShapeDtypeStruct((T, K), jnp.float32)),
    )(tok, Wg)

    ridx, slotpos, blk_eid, blk_valid = _dispatch_host(topi)
    xg = tok[ridx]

    yg = pl.pallas_call(
        _ffn_grouped_kernel,
        grid_spec=pltpu.PrefetchScalarGridSpec(
            num_scalar_prefetch=2,
            grid=(NBLK,),
            in_specs=[
                pl.BlockSpec((BBT, D), lambda b, e_m, v_m: (b, 0)),
                pl.BlockSpec((1, D, H), lambda b, e_m, v_m: (e_m[b], 0, 0)),
                pl.BlockSpec((1, 1, H), lambda b, e_m, v_m: (e_m[b], 0, 0)),
                pl.BlockSpec((1, H, D), lambda b, e_m, v_m: (e_m[b], 0, 0)),
                pl.BlockSpec((1, 1, D), lambda b, e_m, v_m: (e_m[b], 0, 0)),
            ],
            out_specs=pl.BlockSpec((BBT, D), lambda b, e_m, v_m: (b, 0)),
        ),
        out_shape=jax.ShapeDtypeStruct((NPAD, D), jnp.float32),
    )(blk_eid, blk_valid, xg, W1, b1.reshape(E, 1, H), W2,
      b2.reshape(E, 1, D))

    y = (topw[:, 0:1] * yg[slotpos[:, 0]]
         + topw[:, 1:2] * yg[slotpos[:, 1]])
    return y.reshape(B, S, D)
